# Initial kernel scaffold; baseline (speedup 1.0000x reference)
import jax
import jax.numpy as jnp
from jax.experimental import pallas as pl


def kernel(token_ids, weight):
    raise NotImplementedError("write your pallas kernel here")


# speedup vs baseline: 1.8727x; 1.8727x over previous
import functools

import jax
import jax.numpy as jnp
from jax import lax
from jax.experimental import pallas as pl
from jax.experimental.pallas import tpu as pltpu
from jax.experimental.pallas import tpu_sc as plsc

_NC = 2
_NS = 16
_NW = _NC * _NS
_CHUNK = 128
_K = 5
_GROUP = _CHUNK * _K


def _gather_call(B, D, rows_per_w, n_groups):
    mesh = plsc.VectorSubcoreMesh(core_axis_name="c", subcore_axis_name="s")

    @functools.partial(
        pl.kernel,
        out_type=jax.ShapeDtypeStruct((B, D), jnp.float32),
        mesh=mesh,
        compiler_params=pltpu.CompilerParams(use_tc_tiling_on_sc=False),
        scratch_types=[
            pltpu.VMEM((rows_per_w, _CHUNK), jnp.int32),
            pltpu.VMEM((_GROUP, D), jnp.float32),
            pltpu.VMEM((_GROUP, D), jnp.float32),
            pltpu.SemaphoreType.DMA,
            pltpu.SemaphoreType.DMA,
        ],
    )
    def k(idx_hbm, table_hbm, out_hbm, idx_v, buf0, buf1, sem0, sem1):
        wid = lax.axis_index("s") * _NC + lax.axis_index("c")
        idx_base = wid * rows_per_w
        out_base = wid * (rows_per_w * _CHUNK)

        pltpu.sync_copy(idx_hbm.at[pl.ds(idx_base, rows_per_w)], idx_v)

        bufs = (buf0, buf1)
        sems = (sem0, sem1)

        def fire(gg, b):
            for j in range(_K):
                pltpu.async_copy(
                    table_hbm.at[idx_v.at[gg * _K + j]],
                    bufs[b].at[pl.ds(j * _CHUNK, _CHUNK)],
                    sems[b],
                )

        def drain(b):
            pltpu.make_async_copy(
                out_hbm.at[pl.ds(0, _GROUP)], bufs[b], sems[b]
            ).wait()

        fire(0, 0)
        fire(1, 1)

        def step(h, carry):
            g = h * 2
            for b in range(2):
                gg = g + b
                drain(b)
                pltpu.sync_copy(
                    bufs[b], out_hbm.at[pl.ds(out_base + gg * _GROUP, _GROUP)]
                )

                @pl.when(gg + 2 < n_groups)
                def _():
                    fire(gg + 2, b)

            return carry

        lax.fori_loop(0, n_groups // 2, step, 0)

    return k


def kernel(token_ids, weight):
    S0, S1 = token_ids.shape
    B = S0 * S1
    D = weight.shape[1]
    rows_per_w = B // (_NW * _CHUNK)
    n_groups = rows_per_w // _K
    idx = token_ids.astype(jnp.int32).reshape(_NW * rows_per_w, _CHUNK)
    out = _gather_call(B, D, rows_per_w, n_groups)(idx, weight)
    return out.reshape(S0, S1, D)


# speedup vs baseline: 1.8731x; 1.0002x over previous
import functools

import jax
import jax.numpy as jnp
from jax import lax
from jax.experimental import pallas as pl
from jax.experimental.pallas import tpu as pltpu
from jax.experimental.pallas import tpu_sc as plsc

_NC = 2
_NS = 16
_NW = _NC * _NS
_CHUNK = 128
_K = 4
_GROUP = _CHUNK * _K
_NBUF = 3


def _gather_call(B, D, rows_per_w, n_groups):
    mesh = plsc.VectorSubcoreMesh(core_axis_name="c", subcore_axis_name="s")

    @functools.partial(
        pl.kernel,
        out_type=jax.ShapeDtypeStruct((B, D), jnp.float32),
        mesh=mesh,
        compiler_params=pltpu.CompilerParams(use_tc_tiling_on_sc=False),
        scratch_types=[
            pltpu.VMEM((rows_per_w, _CHUNK), jnp.int32),
            [pltpu.VMEM((_GROUP, D), jnp.float32)] * _NBUF,
            [pltpu.SemaphoreType.DMA] * _NBUF,
            [pltpu.SemaphoreType.DMA] * _NBUF,
        ],
    )
    def k(idx_hbm, table_hbm, out_hbm, idx_v, bufs, gsems, wsems):
        wid = lax.axis_index("s") * _NC + lax.axis_index("c")
        idx_base = wid * rows_per_w
        out_base = wid * (rows_per_w * _CHUNK)

        pltpu.sync_copy(idx_hbm.at[pl.ds(idx_base, rows_per_w)], idx_v)

        def fire(gg, b):
            for j in range(_K):
                pltpu.async_copy(
                    table_hbm.at[idx_v.at[gg * _K + j]],
                    bufs[b].at[pl.ds(j * _CHUNK, _CHUNK)],
                    gsems[b],
                )

        def drain(b):
            pltpu.make_async_copy(
                out_hbm.at[pl.ds(0, _GROUP)], bufs[b], gsems[b]
            ).wait()

        def write_start(gg, b):
            pltpu.async_copy(
                bufs[b], out_hbm.at[pl.ds(out_base + gg * _GROUP, _GROUP)],
                wsems[b],
            )

        def write_wait(b):
            pltpu.make_async_copy(
                bufs[b], out_hbm.at[pl.ds(0, _GROUP)], wsems[b]
            ).wait()

        fire(0, 0)
        fire(1, 1)

        n_main = (n_groups - 2) // _NBUF

        def step(h, carry):
            g = h * _NBUF
            for b in range(_NBUF):
                gg = g + b
                drain(b)
                write_start(gg, b)
                nb = (b + 2) % _NBUF

                @pl.when(gg >= 1)
                def _():
                    write_wait(nb)

                fire(gg + 2, nb)
            return carry

        lax.fori_loop(0, n_main, step, 0)

        for gg in range(_NBUF * n_main, n_groups):
            b = gg % _NBUF
            drain(b)
            write_start(gg, b)
        for b in range(_NBUF):
            write_wait(b)

    return k


def kernel(token_ids, weight):
    S0, S1 = token_ids.shape
    B = S0 * S1
    D = weight.shape[1]
    rows_per_w = B // (_NW * _CHUNK)
    n_groups = rows_per_w // _K
    idx = token_ids.astype(jnp.int32).reshape(_NW * rows_per_w, _CHUNK)
    out = _gather_call(B, D, rows_per_w, n_groups)(idx, weight)
    return out.reshape(S0, S1, D)
